# baseline (speedup 1.0000x reference)
import functools

import jax
import jax.numpy as jnp
from jax import lax
from jax.experimental import pallas as pl
from jax.experimental.pallas import tpu as pltpu
from jax.experimental.pallas import tpu_sc as plsc

E = 64
KA = 128
N_VECS = 8192
N_CODES = 8192
BZ = 2048
BK = 4096
NZ = N_VECS // BZ
NK = N_CODES // BK
BIGF = 3.0e38


def _vq_tc_body(z_ref, w_ref, wn_ref, idx_ref, loss_ref, lhs_ref, rhs_ref,
                rv_ref, rj_ref, acc_ref):
    i = pl.program_id(0)
    k = pl.program_id(1)

    @pl.when(i == 0)
    def _():
        wb = w_ref[...]
        wnb = wb / jnp.maximum(
            jnp.sqrt(jnp.sum(wb * wb, axis=1, keepdims=True)), 1e-12)
        rhs_ref[pl.ds(k * BK, BK), :] = jnp.concatenate(
            [-2.0 * wnb, jnp.full((BK, 1), 2.0, jnp.float32),
             jnp.zeros((BK, KA - E - 1), jnp.float32)], axis=1)

    @pl.when(k == 0)
    def _():
        zb = z_ref[...]
        znb = zb / jnp.maximum(
            jnp.sqrt(jnp.sum(zb * zb, axis=1, keepdims=True)), 1e-12)
        lhs_ref[...] = jnp.concatenate(
            [znb, jnp.ones((BZ, 1), jnp.float32),
             jnp.zeros((BZ, KA - E - 1), jnp.float32)], axis=1)
        rv_ref[...] = jnp.full((BZ, 128), BIGF, jnp.float32)
        rj_ref[...] = jnp.zeros((BZ, 128), jnp.float32)

    step = i * NK + k
    srows = N_CODES // (NZ * NK)
    wn_ref[...] = rhs_ref[pl.ds(step * srows, srows), :] * -0.5

    d = lax.dot_general(lhs_ref[...], rhs_ref[pl.ds(k * BK, BK), :],
                        (((1,), (1,)), ((), ())),
                        preferred_element_type=jnp.float32)
    ngrp = BK // 128
    basef = lax.convert_element_type(k * ngrp, jnp.float32)
    rv = rv_ref[...]
    rj = rj_ref[...]
    for j in range(ngrp):
        dj = d[:, j * 128:(j + 1) * 128]
        better = dj < rv
        rv = jnp.minimum(rv, dj)
        rj = jnp.where(better, basef + float(j), rj)
    rv_ref[...] = rv
    rj_ref[...] = rj

    @pl.when(k == NK - 1)
    def _():
        dmin = jnp.min(rv, axis=1, keepdims=True)
        lanef = lax.broadcasted_iota(
            jnp.int32, (BZ, 128), 1).astype(jnp.float32)
        colg = rj * 128.0 + lanef
        amin = jnp.min(jnp.where(rv == dmin, colg, BIGF),
                       axis=1, keepdims=True)
        idx_ref[...] = amin.astype(jnp.int32).reshape(BZ // 128, 128)
        part = jnp.sum(dmin)

        @pl.when(i == 0)
        def _():
            acc_ref[0, 0] = part

        @pl.when(i > 0)
        def _():
            acc_ref[0, 0] = acc_ref[0, 0] + part

        @pl.when(i == NZ - 1)
        def _():
            loss_ref[0, 0] = 1.25 * acc_ref[0, 0] / (N_VECS * E)


def _vq_tc(z2, w):
    return pl.pallas_call(
        _vq_tc_body,
        grid=(NZ, NK),
        in_specs=[
            pl.BlockSpec((BZ, E), lambda i, k: (i, 0)),
            pl.BlockSpec((BK, E), lambda i, k: (k, 0)),
        ],
        out_specs=[
            pl.BlockSpec((N_CODES // (NZ * NK), KA),
                         lambda i, k: (i * NK + k, 0)),
            pl.BlockSpec((BZ // 128, 128), lambda i, k: (i, 0)),
            pl.BlockSpec((1, 1), lambda i, k: (0, 0),
                         memory_space=pltpu.SMEM),
        ],
        out_shape=[
            jax.ShapeDtypeStruct((N_CODES, KA), jnp.float32),
            jax.ShapeDtypeStruct((N_VECS // 128, 128), jnp.int32),
            jax.ShapeDtypeStruct((1, 1), jnp.float32),
        ],
        scratch_shapes=[
            pltpu.VMEM((BZ, KA), jnp.float32),
            pltpu.VMEM((N_CODES, KA), jnp.float32),
            pltpu.VMEM((BZ, 128), jnp.float32),
            pltpu.VMEM((BZ, 128), jnp.float32),
            pltpu.SMEM((1, 1), jnp.float32),
        ],
        compiler_params=pltpu.CompilerParams(
            dimension_semantics=("arbitrary", "arbitrary"),
            vmem_limit_bytes=100 * 1024 * 1024),
    )(z2, w)


def _sc_gather(wn, idx_flat):
    info = plsc.get_sparse_core_info()
    nc, ns = info.num_cores, info.num_subcores
    nw = nc * ns
    per_w = N_VECS // nw
    ch = 128
    nch = per_w // ch
    mesh = plsc.VectorSubcoreMesh(core_axis_name="c", subcore_axis_name="s")

    @functools.partial(
        pl.kernel,
        mesh=mesh,
        out_type=jax.ShapeDtypeStruct((N_VECS, E), jnp.float32),
        scratch_types=[
            pltpu.VMEM((ch,), jnp.int32),
            pltpu.VMEM((ch, KA), jnp.float32),
            pltpu.SemaphoreType.DMA,
        ],
        compiler_params=pltpu.CompilerParams(use_tc_tiling_on_sc=False),
    )
    def gather_k(table_hbm, idx_hbm, out_hbm, idx_v, rows_v, sem):
        wid = lax.axis_index("s") * nc + lax.axis_index("c")
        base = wid * per_w
        for c in range(nch):
            off = base + c * ch
            pltpu.sync_copy(idx_hbm.at[pl.ds(off, ch)], idx_v)
            pltpu.async_copy(table_hbm.at[idx_v], rows_v, sem).wait()
            pltpu.sync_copy(rows_v.at[:, pl.ds(0, E)],
                            out_hbm.at[pl.ds(off, ch)])

    return gather_k(wn, idx_flat)


def kernel(z, W):
    z2 = z.reshape(-1, E)
    wn, idx2, loss2 = _vq_tc(z2, W)
    zq = _sc_gather(wn, idx2.reshape(-1))
    return (zq.reshape(z.shape), loss2[0, 0],
            idx2.reshape(z.shape[:-1]))
